# baseline (speedup 1.0000x reference)
import jax
import jax.numpy as jnp
from jax.experimental import pallas as pl
from jax.experimental.pallas import tpu as pltpu

_BM = 512
_BMM = 1024
_H = 1024
_FF = 4096
_FC = 1024
_NC = _FF // _FC


def _gelu(h):
    return 0.5 * h * (1.0 + jax.lax.erf(h * 0.7071067811865476))


def _norm(x):
    mu = jnp.mean(x, axis=-1, keepdims=True)
    xc = x - mu
    var = jnp.mean(xc * xc, axis=-1, keepdims=True)
    return xc * jax.lax.rsqrt(var + 1e-5)


def _ffn_chunks(ln, x, w1_ref, b1_ref, w2_ref, b2_ref):
    acc = x + b2_ref[...]
    for c in range(_NC):
        sl = slice(c * _FC, (c + 1) * _FC)
        h = (jnp.dot(ln, w1_ref[:, sl], preferred_element_type=jnp.float32)
             + b1_ref[:, sl])
        acc = acc + jnp.dot(_gelu(h), w2_ref[sl, :],
                            preferred_element_type=jnp.float32)
    return acc


def _head_kernel(t_ref, wi_ref, bi_ref,
                 gs_ref, bls_ref, w1s_ref, b1s_ref, w2s_ref, b2s_ref,
                 x_ref, s_ref):
    x = (
        jnp.dot(t_ref[...], wi_ref[...], preferred_element_type=jnp.float32)
        + bi_ref[...]
    )
    x_ref[...] = x
    ln = _norm(x) * gs_ref[...] + bls_ref[...]
    s_ref[...] = _ffn_chunks(ln, x, w1s_ref, b1s_ref, w2s_ref,
                             b2s_ref).astype(jnp.bfloat16)


def _mid_kernel(x_ref, g_ref, bln_ref, w1_ref, b1_ref, w2_ref, b2_ref, o_ref):
    x = x_ref[...]
    ln = _norm(x) * g_ref[...] + bln_ref[...]
    o_ref[...] = _ffn_chunks(ln, x, w1_ref, b1_ref, w2_ref, b2_ref)


def _tail_kernel(d1_ref, g_ref, bln_ref, w1_ref, b1_ref, w2_ref, b2_ref,
                 w_ref, x_ref, s_ref, wo_ref, bo_ref, o_ref):
    d1 = d1_ref[...]
    ln = _norm(d1) * g_ref[...] + bln_ref[...]
    d2 = _ffn_chunks(ln, d1, w1_ref, b1_ref, w2_ref, b2_ref)
    w = w_ref[...]
    fused = (
        w[:, 0:1] * x_ref[...]
        + w[:, 1:2] * s_ref[...].astype(jnp.float32)
        + w[:, 2:3] * d2
    )
    o_ref[...] = (
        jnp.dot(fused, wo_ref[...], preferred_element_type=jnp.float32)
        + bo_ref[...]
    )


def _row_spec(n, bm=_BM):
    return pl.BlockSpec((bm, n), lambda i: (i, 0))


def _full_spec(shape):
    return pl.BlockSpec(shape, lambda i: (0,) * len(shape))


def _params():
    return pltpu.CompilerParams(
        dimension_semantics=("arbitrary",),
        vmem_limit_bytes=64 * 1024 * 1024,
    )


def _ffn_weight_specs():
    return [
        _full_spec((1, _H)),
        _full_spec((1, _H)),
        _full_spec((_H, _FF)),
        _full_spec((1, _FF)),
        _full_spec((_FF, _H)),
        _full_spec((1, _H)),
    ]


def _ffn_weight_args(g, bl, w1, b1, w2, b2):
    return (g.reshape(1, _H), bl.reshape(1, _H), w1, b1.reshape(1, _FF),
            w2, b2.reshape(1, _H))


def kernel(image_tokens, route_probs, W_in, b_in, W_out, b_out,
           shallow_ln_g, shallow_ln_b, shallow_W1, shallow_b1, shallow_W2,
           shallow_b2,
           deep0_ln_g, deep0_ln_b, deep0_W1, deep0_b1, deep0_W2, deep0_b2,
           deep1_ln_g, deep1_ln_b, deep1_W1, deep1_b1, deep1_W2, deep1_b2,
           deep2_ln_g, deep2_ln_b, deep2_W1, deep2_b1, deep2_W2, deep2_b2):
    B, TI, D = image_tokens.shape
    m = B * TI
    grid = (m // _BM,)
    t = image_tokens.reshape(m, D)
    w = route_probs.reshape(m, 3)

    row_f32 = jax.ShapeDtypeStruct((m, _H), jnp.float32)
    row_bf16 = jax.ShapeDtypeStruct((m, _H), jnp.bfloat16)

    x, shallow = pl.pallas_call(
        _head_kernel,
        grid=grid,
        in_specs=(
            [_row_spec(D), _full_spec((D, _H)), _full_spec((1, _H))]
            + _ffn_weight_specs()
        ),
        out_specs=[_row_spec(_H)] * 2,
        out_shape=[row_f32, row_bf16],
        compiler_params=_params(),
    )(
        t, W_in, b_in.reshape(1, _H),
        *_ffn_weight_args(shallow_ln_g, shallow_ln_b, shallow_W1, shallow_b1,
                          shallow_W2, shallow_b2),
    )

    d = x
    for (g, bl, w1, b1, w2, b2) in (
        (deep0_ln_g, deep0_ln_b, deep0_W1, deep0_b1, deep0_W2, deep0_b2),
        (deep1_ln_g, deep1_ln_b, deep1_W1, deep1_b1, deep1_W2, deep1_b2),
    ):
        d = pl.pallas_call(
            _mid_kernel,
            grid=(m // _BMM,),
            in_specs=[_row_spec(_H, _BMM)] + _ffn_weight_specs(),
            out_specs=_row_spec(_H, _BMM),
            out_shape=row_f32,
            compiler_params=_params(),
        )(d, *_ffn_weight_args(g, bl, w1, b1, w2, b2))

    out = pl.pallas_call(
        _tail_kernel,
        grid=grid,
        in_specs=(
            [_row_spec(_H)] + _ffn_weight_specs()
            + [
                pl.BlockSpec((_BM, 3), lambda i: (i, 0)),
                _row_spec(_H),
                _row_spec(_H),
                _full_spec((_H, D)),
                _full_spec((1, D)),
            ]
        ),
        out_specs=_row_spec(D),
        out_shape=jax.ShapeDtypeStruct((m, D), jnp.float32),
        compiler_params=_params(),
    )(
        d,
        *_ffn_weight_args(deep2_ln_g, deep2_ln_b, deep2_W1, deep2_b1,
                          deep2_W2, deep2_b2),
        w, x, shallow, W_out, b_out.reshape(1, D),
    )
    return out.reshape(B, TI, D)


# speedup vs baseline: 1.0103x; 1.0103x over previous
import jax
import jax.numpy as jnp
from jax.experimental import pallas as pl
from jax.experimental.pallas import tpu as pltpu

_BM = 512
_BMM = 512
_H = 1024
_FF = 4096
_FC = 1024
_NC = _FF // _FC


def _gelu(h):
    return 0.5 * h * (1.0 + jax.lax.erf(h * 0.7071067811865476))


def _norm(x):
    mu = jnp.mean(x, axis=-1, keepdims=True)
    xc = x - mu
    var = jnp.mean(xc * xc, axis=-1, keepdims=True)
    return xc * jax.lax.rsqrt(var + 1e-5)


def _ffn_chunks(ln, x, w1_ref, b1_ref, w2_ref, b2_ref):
    acc = x + b2_ref[...]
    for c in range(_NC):
        sl = slice(c * _FC, (c + 1) * _FC)
        h = (jnp.dot(ln, w1_ref[:, sl], preferred_element_type=jnp.float32)
             + b1_ref[:, sl])
        acc = acc + jnp.dot(_gelu(h), w2_ref[sl, :],
                            preferred_element_type=jnp.float32)
    return acc


def _head_kernel(t_ref, wi_ref, bi_ref,
                 gs_ref, bls_ref, w1s_ref, b1s_ref, w2s_ref, b2s_ref,
                 x_ref, s_ref):
    x = (
        jnp.dot(t_ref[...], wi_ref[...], preferred_element_type=jnp.float32)
        + bi_ref[...]
    )
    x_ref[...] = x
    ln = _norm(x) * gs_ref[...] + bls_ref[...]
    s_ref[...] = _ffn_chunks(ln, x, w1s_ref, b1s_ref, w2s_ref,
                             b2s_ref).astype(jnp.bfloat16)


def _mid_kernel(x_ref, g_ref, bln_ref, w1_ref, b1_ref, w2_ref, b2_ref, o_ref):
    x = x_ref[...]
    ln = _norm(x) * g_ref[...] + bln_ref[...]
    o_ref[...] = _ffn_chunks(ln, x, w1_ref, b1_ref, w2_ref, b2_ref)


def _tail_kernel(d1_ref, g_ref, bln_ref, w1_ref, b1_ref, w2_ref, b2_ref,
                 w_ref, x_ref, s_ref, wo_ref, bo_ref, o_ref):
    d1 = d1_ref[...]
    ln = _norm(d1) * g_ref[...] + bln_ref[...]
    d2 = _ffn_chunks(ln, d1, w1_ref, b1_ref, w2_ref, b2_ref)
    w = w_ref[...]
    fused = (
        w[:, 0:1] * x_ref[...]
        + w[:, 1:2] * s_ref[...].astype(jnp.float32)
        + w[:, 2:3] * d2
    )
    o_ref[...] = (
        jnp.dot(fused, wo_ref[...], preferred_element_type=jnp.float32)
        + bo_ref[...]
    )


def _row_spec(n, bm=_BM):
    return pl.BlockSpec((bm, n), lambda i: (i, 0))


def _full_spec(shape):
    return pl.BlockSpec(shape, lambda i: (0,) * len(shape))


def _params():
    return pltpu.CompilerParams(
        dimension_semantics=("arbitrary",),
        vmem_limit_bytes=64 * 1024 * 1024,
    )


def _ffn_weight_specs():
    return [
        _full_spec((1, _H)),
        _full_spec((1, _H)),
        _full_spec((_H, _FF)),
        _full_spec((1, _FF)),
        _full_spec((_FF, _H)),
        _full_spec((1, _H)),
    ]


def _ffn_weight_args(g, bl, w1, b1, w2, b2):
    return (g.reshape(1, _H), bl.reshape(1, _H), w1, b1.reshape(1, _FF),
            w2, b2.reshape(1, _H))


def kernel(image_tokens, route_probs, W_in, b_in, W_out, b_out,
           shallow_ln_g, shallow_ln_b, shallow_W1, shallow_b1, shallow_W2,
           shallow_b2,
           deep0_ln_g, deep0_ln_b, deep0_W1, deep0_b1, deep0_W2, deep0_b2,
           deep1_ln_g, deep1_ln_b, deep1_W1, deep1_b1, deep1_W2, deep1_b2,
           deep2_ln_g, deep2_ln_b, deep2_W1, deep2_b1, deep2_W2, deep2_b2):
    B, TI, D = image_tokens.shape
    m = B * TI
    grid = (m // _BM,)
    t = image_tokens.reshape(m, D)
    w = route_probs.reshape(m, 3)

    row_f32 = jax.ShapeDtypeStruct((m, _H), jnp.float32)
    row_bf16 = jax.ShapeDtypeStruct((m, _H), jnp.bfloat16)

    x, shallow = pl.pallas_call(
        _head_kernel,
        grid=grid,
        in_specs=(
            [_row_spec(D), _full_spec((D, _H)), _full_spec((1, _H))]
            + _ffn_weight_specs()
        ),
        out_specs=[_row_spec(_H)] * 2,
        out_shape=[row_f32, row_bf16],
        compiler_params=_params(),
    )(
        t, W_in, b_in.reshape(1, _H),
        *_ffn_weight_args(shallow_ln_g, shallow_ln_b, shallow_W1, shallow_b1,
                          shallow_W2, shallow_b2),
    )

    d = x
    for (g, bl, w1, b1, w2, b2) in (
        (deep0_ln_g, deep0_ln_b, deep0_W1, deep0_b1, deep0_W2, deep0_b2),
        (deep1_ln_g, deep1_ln_b, deep1_W1, deep1_b1, deep1_W2, deep1_b2),
    ):
        d = pl.pallas_call(
            _mid_kernel,
            grid=(m // _BMM,),
            in_specs=[_row_spec(_H, _BMM)] + _ffn_weight_specs(),
            out_specs=_row_spec(_H, _BMM),
            out_shape=row_f32,
            compiler_params=_params(),
        )(d, *_ffn_weight_args(g, bl, w1, b1, w2, b2))

    out = pl.pallas_call(
        _tail_kernel,
        grid=grid,
        in_specs=(
            [_row_spec(_H)] + _ffn_weight_specs()
            + [
                pl.BlockSpec((_BM, 3), lambda i: (i, 0)),
                _row_spec(_H),
                _row_spec(_H),
                _full_spec((_H, D)),
                _full_spec((1, D)),
            ]
        ),
        out_specs=_row_spec(D),
        out_shape=jax.ShapeDtypeStruct((m, D), jnp.float32),
        compiler_params=_params(),
    )(
        d,
        *_ffn_weight_args(deep2_ln_g, deep2_ln_b, deep2_W1, deep2_b1,
                          deep2_W2, deep2_b2),
        w, x, shallow, W_out, b_out.reshape(1, D),
    )
    return out.reshape(B, TI, D)


# speedup vs baseline: 1.0167x; 1.0063x over previous
import jax
import jax.numpy as jnp
from jax.experimental import pallas as pl
from jax.experimental.pallas import tpu as pltpu

_BM = 512
_H = 1024
_FF = 4096
_FC = 2048
_NC = _FF // _FC


def _gelu(h):
    return 0.5 * h * (1.0 + jax.lax.erf(h * 0.7071067811865476))


def _norm(x):
    mu = jnp.mean(x, axis=-1, keepdims=True)
    xc = x - mu
    var = jnp.mean(xc * xc, axis=-1, keepdims=True)
    return xc * jax.lax.rsqrt(var + 1e-5)


def _ffn_chunks(ln, x, w1_ref, b1_ref, w2_ref, b2_ref):
    acc = x + b2_ref[...]
    for c in range(_NC):
        sl = slice(c * _FC, (c + 1) * _FC)
        h = (jnp.dot(ln, w1_ref[:, sl], preferred_element_type=jnp.float32)
             + b1_ref[:, sl])
        acc = acc + jnp.dot(_gelu(h), w2_ref[sl, :],
                            preferred_element_type=jnp.float32)
    return acc


def _head_kernel(t_ref, wi_ref, bi_ref,
                 gs_ref, bls_ref, w1s_ref, b1s_ref, w2s_ref, b2s_ref,
                 x_ref, s_ref):
    x = (
        jnp.dot(t_ref[...], wi_ref[...], preferred_element_type=jnp.float32)
        + bi_ref[...]
    )
    x_ref[...] = x
    ln = _norm(x) * gs_ref[...] + bls_ref[...]
    s_ref[...] = _ffn_chunks(ln, x, w1s_ref, b1s_ref, w2s_ref,
                             b2s_ref).astype(jnp.bfloat16)


def _mid_kernel(x_ref, g_ref, bln_ref, w1_ref, b1_ref, w2_ref, b2_ref, o_ref):
    x = x_ref[...]
    ln = _norm(x) * g_ref[...] + bln_ref[...]
    o_ref[...] = _ffn_chunks(ln, x, w1_ref, b1_ref, w2_ref, b2_ref)


def _tail_kernel(d1_ref, g_ref, bln_ref, w1_ref, b1_ref, w2_ref, b2_ref,
                 w_ref, x_ref, s_ref, wo_ref, bo_ref, o_ref):
    d1 = d1_ref[...]
    ln = _norm(d1) * g_ref[...] + bln_ref[...]
    d2 = _ffn_chunks(ln, d1, w1_ref, b1_ref, w2_ref, b2_ref)
    w = w_ref[...]
    fused = (
        w[:, 0:1] * x_ref[...]
        + w[:, 1:2] * s_ref[...].astype(jnp.float32)
        + w[:, 2:3] * d2
    )
    o_ref[...] = (
        jnp.dot(fused, wo_ref[...], preferred_element_type=jnp.float32)
        + bo_ref[...]
    )


def _row_spec(n):
    return pl.BlockSpec((_BM, n), lambda i: (i, 0))


def _full_spec(shape):
    return pl.BlockSpec(shape, lambda i: (0,) * len(shape))


def _params():
    return pltpu.CompilerParams(
        dimension_semantics=("parallel",),
        vmem_limit_bytes=64 * 1024 * 1024,
    )


def _ffn_weight_specs():
    return [
        _full_spec((1, _H)),
        _full_spec((1, _H)),
        _full_spec((_H, _FF)),
        _full_spec((1, _FF)),
        _full_spec((_FF, _H)),
        _full_spec((1, _H)),
    ]


def _ffn_weight_args(g, bl, w1, b1, w2, b2):
    return (g.reshape(1, _H), bl.reshape(1, _H), w1, b1.reshape(1, _FF),
            w2, b2.reshape(1, _H))


def kernel(image_tokens, route_probs, W_in, b_in, W_out, b_out,
           shallow_ln_g, shallow_ln_b, shallow_W1, shallow_b1, shallow_W2,
           shallow_b2,
           deep0_ln_g, deep0_ln_b, deep0_W1, deep0_b1, deep0_W2, deep0_b2,
           deep1_ln_g, deep1_ln_b, deep1_W1, deep1_b1, deep1_W2, deep1_b2,
           deep2_ln_g, deep2_ln_b, deep2_W1, deep2_b1, deep2_W2, deep2_b2):
    B, TI, D = image_tokens.shape
    m = B * TI
    grid = (m // _BM,)
    t = image_tokens.reshape(m, D)
    w = route_probs.reshape(m, 3)

    row_f32 = jax.ShapeDtypeStruct((m, _H), jnp.float32)
    row_bf16 = jax.ShapeDtypeStruct((m, _H), jnp.bfloat16)

    x, shallow = pl.pallas_call(
        _head_kernel,
        grid=grid,
        in_specs=(
            [_row_spec(D), _full_spec((D, _H)), _full_spec((1, _H))]
            + _ffn_weight_specs()
        ),
        out_specs=[_row_spec(_H)] * 2,
        out_shape=[row_f32, row_bf16],
        compiler_params=_params(),
    )(
        t, W_in, b_in.reshape(1, _H),
        *_ffn_weight_args(shallow_ln_g, shallow_ln_b, shallow_W1, shallow_b1,
                          shallow_W2, shallow_b2),
    )

    d = x
    for (g, bl, w1, b1, w2, b2) in (
        (deep0_ln_g, deep0_ln_b, deep0_W1, deep0_b1, deep0_W2, deep0_b2),
        (deep1_ln_g, deep1_ln_b, deep1_W1, deep1_b1, deep1_W2, deep1_b2),
    ):
        d = pl.pallas_call(
            _mid_kernel,
            grid=grid,
            in_specs=[_row_spec(_H)] + _ffn_weight_specs(),
            out_specs=_row_spec(_H),
            out_shape=row_f32,
            compiler_params=_params(),
        )(d, *_ffn_weight_args(g, bl, w1, b1, w2, b2))

    out = pl.pallas_call(
        _tail_kernel,
        grid=grid,
        in_specs=(
            [_row_spec(_H)] + _ffn_weight_specs()
            + [
                pl.BlockSpec((_BM, 3), lambda i: (i, 0)),
                _row_spec(_H),
                _row_spec(_H),
                _full_spec((_H, D)),
                _full_spec((1, D)),
            ]
        ),
        out_specs=_row_spec(D),
        out_shape=jax.ShapeDtypeStruct((m, D), jnp.float32),
        compiler_params=_params(),
    )(
        d,
        *_ffn_weight_args(deep2_ln_g, deep2_ln_b, deep2_W1, deep2_b1,
                          deep2_W2, deep2_b2),
        w, x, shallow, W_out, b_out.reshape(1, D),
    )
    return out.reshape(B, TI, D)
